# Initial kernel scaffold; baseline (speedup 1.0000x reference)
import functools
import math

import jax
import jax.numpy as jnp
from jax import lax
from jax.experimental import pallas as pl
from jax.experimental.pallas import tpu as pltpu
from jax.experimental.pallas import tpu_sc as plsc

_SIZE = 32000
_PAD = 0
_SMOOTHING = 0.1
_CONF = 1.0 - _SMOOTHING
_N = 4096
_S = _SMOOTHING / (_SIZE - 2)
_K = (_SIZE - 2) * _S * math.log(_S) + _CONF * math.log(_CONF)

_LANES = 16
_NC = 2
_NS = 16
_NW = _NC * _NS
_RPW = _N // _NW

_TC_BLOCK_ROWS = 128


def _tc_body(x_ref, t_ref, sum_ref, cnt_ref):
    i = pl.program_id(0)
    x = x_ref[...]
    t = t_ref[...]
    m = t != _PAD
    rs = jnp.sum(x, axis=1, keepdims=True)
    x0 = x[:, 0:1]
    part = jnp.sum(jnp.where(m, _S * x0 - _S * rs, 0.0))
    cnt = jnp.sum(m.astype(jnp.float32))

    @pl.when(i == 0)
    def _():
        sum_ref[0, 0] = 0.0
        cnt_ref[0, 0] = 0.0

    sum_ref[0, 0] += part
    cnt_ref[0, 0] += cnt


def _tc_call(x, t2d):
    grid = _N // _TC_BLOCK_ROWS
    return pl.pallas_call(
        _tc_body,
        grid=(grid,),
        in_specs=[
            pl.BlockSpec((_TC_BLOCK_ROWS, _SIZE), lambda i: (i, 0)),
            pl.BlockSpec((_TC_BLOCK_ROWS, 1), lambda i: (i, 0)),
        ],
        out_specs=[
            pl.BlockSpec(memory_space=pltpu.SMEM),
            pl.BlockSpec(memory_space=pltpu.SMEM),
        ],
        out_shape=[
            jax.ShapeDtypeStruct((1, 1), jnp.float32),
            jax.ShapeDtypeStruct((1, 1), jnp.float32),
        ],
    )(x, t2d)


@functools.partial(
    pl.kernel,
    out_type=jax.ShapeDtypeStruct((_NW, _LANES), jnp.float32),
    mesh=plsc.VectorSubcoreMesh(
        core_axis_name="c", subcore_axis_name="s",
        num_cores=_NC, num_subcores=_NS,
    ),
    scratch_types=[
        pltpu.VMEM((_RPW,), jnp.int32),
        pltpu.VMEM((_RPW,), jnp.int32),
        pltpu.VMEM((_RPW, _LANES), jnp.float32),
        pltpu.VMEM((_LANES,), jnp.float32),
        pltpu.SemaphoreType.DMA,
    ],
)
def _sc_gather(x16_hbm, tgt_hbm, out_hbm, tgt_v, ridx_v, rows_v, acc_v, sem):
    wid = lax.axis_index("s") * _NC + lax.axis_index("c")
    base = wid * _RPW
    pltpu.sync_copy(tgt_hbm.at[pl.ds(base, _RPW)], tgt_v)

    lanes = lax.iota(jnp.int32, _LANES)
    rows_per_chunk = _SIZE // _LANES
    for j in range(_RPW // _LANES):
        t = tgt_v[pl.ds(j * _LANES, _LANES)]
        ivec = base + j * _LANES + lanes
        ridx_v[pl.ds(j * _LANES, _LANES)] = ivec * rows_per_chunk + (t >> 4)

    pltpu.async_copy(x16_hbm.at[ridx_v], rows_v, sem).wait()

    acc = jnp.zeros((_LANES,), jnp.float32)
    for j in range(_RPW // _LANES):
        t = tgt_v[pl.ds(j * _LANES, _LANES)]
        vals = plsc.load_gather(rows_v, [j * _LANES + lanes, t & (_LANES - 1)])
        acc = acc + jnp.where(t != _PAD, vals, 0.0)
    acc_v[...] = acc
    pltpu.sync_copy(acc_v, out_hbm.at[wid])


def kernel(x, target, mu, logvar, beta):
    del mu, logvar, beta
    tgt = target.astype(jnp.int32)
    tc_sum, tc_cnt = _tc_call(x, tgt.reshape(_N, 1))
    sc_out = _sc_gather(x.reshape(-1, _LANES), tgt)
    s3 = jnp.sum(sc_out)
    total = tc_cnt[0, 0] * _K + tc_sum[0, 0] + (_S - _CONF) * s3
    return total / _N


# speedup vs baseline: 2.4459x; 2.4459x over previous
import functools
import math

import jax
import jax.numpy as jnp
from jax import lax
from jax.experimental import pallas as pl
from jax.experimental.pallas import tpu as pltpu
from jax.experimental.pallas import tpu_sc as plsc

_SIZE = 32000
_PAD = 0
_SMOOTHING = 0.1
_CONF = 1.0 - _SMOOTHING
_N = 4096
_S = _SMOOTHING / (_SIZE - 2)
_K = (_SIZE - 2) * _S * math.log(_S) + _CONF * math.log(_CONF)

_LANES = 16
_NC = 2
_NS = 16
_NW = _NC * _NS
_RPW = _N // _NW
_CHUNKS = _SIZE // _LANES

_SEGV = _CHUNKS // 2
_NSEG = _RPW * 2
_NBUF = 4


def _cnt_body(t_ref, cnt_ref):
    cnt_ref[0, 0] = jnp.sum((t_ref[...] != _PAD).astype(jnp.float32))


def _cnt_call(t2d):
    return pl.pallas_call(
        _cnt_body,
        in_specs=[pl.BlockSpec(memory_space=pltpu.VMEM)],
        out_specs=pl.BlockSpec(memory_space=pltpu.SMEM),
        out_shape=jax.ShapeDtypeStruct((1, 1), jnp.float32),
    )(t2d)


def _lane_pick(row, lvec):
    return lax.gather(
        row, lvec[:, None],
        lax.GatherDimensionNumbers(
            offset_dims=(), collapsed_slice_dims=(0,),
            start_index_map=(0,)),
        slice_sizes=(1,),
        mode=lax.GatherScatterMode.PROMISE_IN_BOUNDS)


@functools.partial(
    pl.kernel,
    out_type=[
        jax.ShapeDtypeStruct((_NW, _LANES), jnp.float32),
        jax.ShapeDtypeStruct((_NW, _LANES), jnp.float32),
    ],
    mesh=plsc.VectorSubcoreMesh(
        core_axis_name="c", subcore_axis_name="s",
        num_cores=_NC, num_subcores=_NS,
    ),
    compiler_params=pltpu.CompilerParams(use_tc_tiling_on_sc=False),
    scratch_types=[
        pltpu.VMEM((_RPW,), jnp.int32),
        pltpu.VMEM((_RPW,), jnp.int32),
        pltpu.VMEM((_RPW, _LANES), jnp.float32),
        [pltpu.VMEM((_SEGV, _LANES), jnp.float32) for _ in range(_NBUF)],
        pltpu.VMEM((_LANES,), jnp.float32),
        pltpu.VMEM((_LANES,), jnp.float32),
        pltpu.SemaphoreType.DMA,
        [pltpu.SemaphoreType.DMA for _ in range(_NBUF)],
    ],
)
def _sc_kernel(x16_hbm, tgt_hbm, gout_hbm, rout_hbm,
               tgt_v, ridx_v, rows_v, bufs, gacc_v, racc_v, gsem, sems):
    wid = lax.axis_index("s") * _NC + lax.axis_index("c")
    lanes = lax.iota(jnp.int32, _LANES)
    base = wid * _RPW

    pltpu.sync_copy(tgt_hbm.at[pl.ds(base, _RPW)], tgt_v)

    for j in range(_RPW // _LANES):
        t = tgt_v[pl.ds(j * _LANES, _LANES)]
        ivec = base + j * _LANES + lanes
        ridx_v[pl.ds(j * _LANES, _LANES)] = ivec * _CHUNKS + (t >> 4)
    gather_dma = pltpu.async_copy(x16_hbm.at[ridx_v], rows_v, gsem)

    vbase = base * _CHUNKS

    def seg_dma(s, b):
        return pltpu.make_async_copy(
            x16_hbm.at[pl.ds(vbase + s * _SEGV, _SEGV), :], bufs[b], sems[b])

    for b in range(_NBUF):
        seg_dma(b, b).start()

    def seg_sum(buf):
        def chunk(c, accs):
            a0, a1, a2, a3 = accs
            o = c * 8
            a0 = a0 + buf[o, :] + buf[o + 4, :]
            a1 = a1 + buf[o + 1, :] + buf[o + 5, :]
            a2 = a2 + buf[o + 2, :] + buf[o + 6, :]
            a3 = a3 + buf[o + 3, :] + buf[o + 7, :]
            return a0, a1, a2, a3
        z = jnp.zeros((_LANES,), jnp.float32)
        a0, a1, a2, a3 = lax.fori_loop(0, _SEGV // 8, chunk, (z, z, z, z))
        return (a0 + a1) + (a2 + a3)

    def group(gi, carry):
        racc, x0acc = carry
        for b in range(_NBUF):
            s = gi * _NBUF + b
            seg_dma(s, b).wait()
            svec = seg_sum(bufs[b])
            t = tgt_v[pl.ds(((s >> 5) << 4), _LANES)]
            t_r = _lane_pick(
                t, jnp.broadcast_to((s >> 1) & (_LANES - 1), (_LANES,)))
            mf = jnp.minimum(jnp.abs(t_r), 1).astype(jnp.float32)
            racc = racc + mf * svec
            x0f = (1 - (s & 1)).astype(jnp.float32)
            x0acc = x0acc + (mf * x0f) * jnp.where(
                lanes == 0, bufs[b][0, :], 0.0)

            @pl.when(s + _NBUF < _NSEG)
            def _():
                seg_dma(s + _NBUF, b).start()
        return racc, x0acc

    z16 = jnp.zeros((_LANES,), jnp.float32)
    racc, x0acc = lax.fori_loop(0, _NSEG // _NBUF, group, (z16, z16))
    racc_v[...] = _S * x0acc - _S * racc
    pltpu.sync_copy(racc_v, rout_hbm.at[wid])

    gather_dma.wait()
    gacc = jnp.zeros((_LANES,), jnp.float32)
    for c in range(_RPW // _LANES):
        t = tgt_v[pl.ds(c * _LANES, _LANES)]
        lvec = t & (_LANES - 1)
        keep = t != _PAD
        for j in range(_LANES):
            g = _lane_pick(rows_v[c * _LANES + j, :], lvec)
            gacc = gacc + jnp.where((lanes == j) & keep, g, 0.0)
    gacc_v[...] = gacc
    pltpu.sync_copy(gacc_v, gout_hbm.at[wid])


def kernel(x, target, mu, logvar, beta):
    del mu, logvar, beta
    tgt = target.astype(jnp.int32)
    g_out, r_out = _sc_kernel(x.reshape(-1, _LANES), tgt)
    cnt = _cnt_call(tgt.reshape(_N, 1))
    total = (cnt[0, 0] * _K + jnp.sum(r_out)
             + (_S - _CONF) * jnp.sum(g_out))
    return total / _N


# speedup vs baseline: 6.5515x; 2.6786x over previous
import functools
import math

import jax
import jax.numpy as jnp
from jax import lax
from jax.experimental import pallas as pl
from jax.experimental.pallas import tpu as pltpu
from jax.experimental.pallas import tpu_sc as plsc

_SIZE = 32000
_PAD = 0
_SMOOTHING = 0.1
_CONF = 1.0 - _SMOOTHING
_N = 4096
_S = _SMOOTHING / (_SIZE - 2)
_K = (_SIZE - 2) * _S * math.log(_S) + _CONF * math.log(_CONF)

_LANES = 16
_NC = 2
_NS = 16
_NW = _NC * _NS
_RPW = _N // _NW
_CHUNKS = _SIZE // _LANES

_VPW = _RPW * _CHUNKS
_SEGV = 1600
_NSEG = _VPW // _SEGV
_SEG_PER_GROUP = 10
_NBUF = 4


def _cnt_body(t_ref, cnt_ref):
    cnt_ref[0, 0] = jnp.sum((t_ref[...] != _PAD).astype(jnp.float32))


def _cnt_call(t2d):
    return pl.pallas_call(
        _cnt_body,
        in_specs=[pl.BlockSpec(memory_space=pltpu.VMEM)],
        out_specs=pl.BlockSpec(memory_space=pltpu.SMEM),
        out_shape=jax.ShapeDtypeStruct((1, 1), jnp.float32),
    )(t2d)


def _lane_pick(row, lvec):
    return lax.gather(
        row, lvec[:, None],
        lax.GatherDimensionNumbers(
            offset_dims=(), collapsed_slice_dims=(0,),
            start_index_map=(0,)),
        slice_sizes=(1,),
        mode=lax.GatherScatterMode.PROMISE_IN_BOUNDS)


@functools.partial(
    pl.kernel,
    out_type=[
        jax.ShapeDtypeStruct((_NW, _LANES), jnp.float32),
        jax.ShapeDtypeStruct((_NW, _LANES), jnp.float32),
    ],
    mesh=plsc.VectorSubcoreMesh(
        core_axis_name="c", subcore_axis_name="s",
        num_cores=_NC, num_subcores=_NS,
    ),
    compiler_params=pltpu.CompilerParams(use_tc_tiling_on_sc=False),
    scratch_types=[
        pltpu.VMEM((_RPW,), jnp.int32),
        pltpu.VMEM((_RPW,), jnp.int32),
        pltpu.VMEM((_RPW, _LANES), jnp.float32),
        [pltpu.VMEM((_SEGV, _LANES), jnp.float32) for _ in range(_NBUF)],
        pltpu.VMEM((_LANES,), jnp.float32),
        pltpu.VMEM((_LANES,), jnp.float32),
        pltpu.SemaphoreType.DMA,
        [pltpu.SemaphoreType.DMA for _ in range(_NBUF)],
    ],
)
def _sc_kernel(x16_hbm, tgt_hbm, gout_hbm, rout_hbm,
               tgt_v, ridx_v, rows_v, bufs, gacc_v, racc_v, gsem, sems):
    wid = lax.axis_index("s") * _NC + lax.axis_index("c")
    lanes = lax.iota(jnp.int32, _LANES)
    base = wid * _RPW

    pltpu.sync_copy(tgt_hbm.at[pl.ds(base, _RPW)], tgt_v)

    for j in range(_RPW // _LANES):
        t = tgt_v[pl.ds(j * _LANES, _LANES)]
        ivec = base + j * _LANES + lanes
        stripe = ((ivec >> 3) * 250 + (t >> 7)) * 8 + (ivec & 7)
        ridx_v[pl.ds(j * _LANES, _LANES)] = stripe * 8 + ((t >> 4) & 7)
    gather_dma = pltpu.async_copy(x16_hbm.at[ridx_v], rows_v, gsem)

    vbase = wid * _VPW

    def seg_dma(s, b):
        return pltpu.make_async_copy(
            x16_hbm.at[pl.ds(vbase + s * _SEGV, _SEGV), :], bufs[b], sems[b])

    for b in range(_NBUF):
        seg_dma(b, b).start()

    def seg_body(s, b, carry):
        racc, x0acc = carry
        buf = bufs[b]

        def tile(ti, accs):
            o = ti * 64
            out = []
            for c in range(8):
                a = accs[c]
                for q in range(8):
                    a = a + buf[o + c * 8 + q, :]
                out.append(a)
            return tuple(out)

        z = jnp.zeros((_LANES,), jnp.float32)
        accs = lax.fori_loop(0, _SEGV // 64, tile, (z,) * 8)

        lr0 = (s // _SEG_PER_GROUP) * 8
        tg = tgt_v[pl.ds((lr0 >> 4) << 4, _LANES)]
        x0f = jnp.where(s % _SEG_PER_GROUP == 0, 1.0, 0.0)
        for c in range(8):
            t_r = _lane_pick(
                tg, jnp.broadcast_to((lr0 + c) & (_LANES - 1), (_LANES,)))
            mf = jnp.minimum(jnp.abs(t_r), 1).astype(jnp.float32)
            racc = racc + mf * accs[c]
            x0acc = x0acc + (mf * x0f) * jnp.where(
                lanes == 0, buf[c * 8, :], 0.0)
        return racc, x0acc

    def group(gi, carry):
        for b in range(_NBUF):
            s = gi * _NBUF + b
            seg_dma(s, b).wait()
            carry = seg_body(s, b, carry)

            @pl.when(s + _NBUF < _NSEG)
            def _():
                seg_dma(s + _NBUF, b).start()
        return carry

    z16 = jnp.zeros((_LANES,), jnp.float32)
    racc, x0acc = lax.fori_loop(0, _NSEG // _NBUF, group, (z16, z16))
    racc_v[...] = _S * x0acc - _S * racc
    pltpu.sync_copy(racc_v, rout_hbm.at[wid])

    gather_dma.wait()
    gacc = jnp.zeros((_LANES,), jnp.float32)
    for c in range(_RPW // _LANES):
        t = tgt_v[pl.ds(c * _LANES, _LANES)]
        lvec = t & (_LANES - 1)
        keep = t != _PAD
        for j in range(_LANES):
            g = _lane_pick(rows_v[c * _LANES + j, :], lvec)
            gacc = gacc + jnp.where((lanes == j) & keep, g, 0.0)
    gacc_v[...] = gacc
    pltpu.sync_copy(gacc_v, gout_hbm.at[wid])


def kernel(x, target, mu, logvar, beta):
    del mu, logvar, beta
    tgt = target.astype(jnp.int32)
    x4 = x.reshape(512, 8, 250, 128).transpose(0, 2, 1, 3).reshape(-1, _LANES)
    g_out, r_out = _sc_kernel(x4, tgt)
    cnt = _cnt_call(tgt.reshape(_N, 1))
    total = (cnt[0, 0] * _K + jnp.sum(r_out)
             + (_S - _CONF) * jnp.sum(g_out))
    return total / _N


# speedup vs baseline: 6.8420x; 1.0443x over previous
import functools
import math

import jax
import jax.numpy as jnp
from jax import lax
from jax.experimental import pallas as pl
from jax.experimental.pallas import tpu as pltpu
from jax.experimental.pallas import tpu_sc as plsc

_SIZE = 32000
_PAD = 0
_SMOOTHING = 0.1
_CONF = 1.0 - _SMOOTHING
_N = 4096
_S = _SMOOTHING / (_SIZE - 2)
_K = (_SIZE - 2) * _S * math.log(_S) + _CONF * math.log(_CONF)

_LANES = 16
_NC = 2
_NS = 16
_NW = _NC * _NS
_RPW = _N // _NW
_CHUNKS = _SIZE // _LANES

_TC_ROWS = 1024
_RSROWS = (_N - _TC_ROWS) // _NW
_VPW = _RSROWS * _CHUNKS
_SEGV = 1600
_NSEG = _VPW // _SEGV
_SEG_PER_GROUP = 10
_NBUF = 4

_TC_BLOCK_ROWS = 64
_TC_NBUF = 6
_TC_NBLOCKS = _TC_ROWS // _TC_BLOCK_ROWS


def _tc_body(x_hbm, t_ref, sum_ref, cnt_ref, bufs, sems):
    def start(i):
        b = i % _TC_NBUF
        pltpu.make_async_copy(
            x_hbm.at[pl.ds(i * _TC_BLOCK_ROWS, _TC_BLOCK_ROWS), :],
            bufs.at[b], sems.at[b]).start()

    for i in range(_TC_NBUF):
        start(i)

    acc = jnp.float32(0.0)
    for i in range(_TC_NBLOCKS):
        b = i % _TC_NBUF
        pltpu.make_async_copy(
            x_hbm.at[pl.ds(i * _TC_BLOCK_ROWS, _TC_BLOCK_ROWS), :],
            bufs.at[b], sems.at[b]).wait()
        xb = bufs[b]
        t = t_ref[pl.ds(i * _TC_BLOCK_ROWS, _TC_BLOCK_ROWS), :]
        m = t != _PAD
        rs = jnp.sum(xb, axis=1, keepdims=True)
        x0 = xb[:, 0:1]
        acc = acc + jnp.sum(jnp.where(m, _S * x0 - _S * rs, 0.0))
        if i + _TC_NBUF < _TC_NBLOCKS:
            start(i + _TC_NBUF)

    cnt = jnp.sum((t_ref[...] != _PAD).astype(jnp.float32))
    sum_ref[0, 0] = acc
    cnt_ref[0, 0] = cnt


def _tc_call(x, t2d):
    return pl.pallas_call(
        _tc_body,
        in_specs=[
            pl.BlockSpec(memory_space=pl.ANY),
            pl.BlockSpec(memory_space=pltpu.VMEM),
        ],
        out_specs=[
            pl.BlockSpec(memory_space=pltpu.SMEM),
            pl.BlockSpec(memory_space=pltpu.SMEM),
        ],
        out_shape=[
            jax.ShapeDtypeStruct((1, 1), jnp.float32),
            jax.ShapeDtypeStruct((1, 1), jnp.float32),
        ],
        scratch_shapes=[
            pltpu.VMEM((_TC_NBUF, _TC_BLOCK_ROWS, _SIZE), jnp.float32),
            pltpu.SemaphoreType.DMA((_TC_NBUF,)),
        ],
    )(x, t2d)


def _lane_pick(row, lvec):
    return lax.gather(
        row, lvec[:, None],
        lax.GatherDimensionNumbers(
            offset_dims=(), collapsed_slice_dims=(0,),
            start_index_map=(0,)),
        slice_sizes=(1,),
        mode=lax.GatherScatterMode.PROMISE_IN_BOUNDS)


@functools.partial(
    pl.kernel,
    out_type=[
        jax.ShapeDtypeStruct((_NW, _LANES), jnp.float32),
        jax.ShapeDtypeStruct((_NW, _LANES), jnp.float32),
    ],
    mesh=plsc.VectorSubcoreMesh(
        core_axis_name="c", subcore_axis_name="s",
        num_cores=_NC, num_subcores=_NS,
    ),
    compiler_params=pltpu.CompilerParams(use_tc_tiling_on_sc=False),
    scratch_types=[
        pltpu.VMEM((_RPW,), jnp.int32),
        pltpu.VMEM((_RSROWS,), jnp.int32),
        pltpu.VMEM((_RPW,), jnp.int32),
        pltpu.VMEM((_RPW, _LANES), jnp.float32),
        [pltpu.VMEM((_SEGV, _LANES), jnp.float32) for _ in range(_NBUF)],
        pltpu.VMEM((_LANES,), jnp.float32),
        pltpu.VMEM((_LANES,), jnp.float32),
        pltpu.SemaphoreType.DMA,
        [pltpu.SemaphoreType.DMA for _ in range(_NBUF)],
    ],
)
def _sc_kernel(x16_hbm, tgt_hbm, gout_hbm, rout_hbm,
               tgt_v, tgt2_v, ridx_v, rows_v, bufs, gacc_v, racc_v,
               gsem, sems):
    wid = lax.axis_index("s") * _NC + lax.axis_index("c")
    lanes = lax.iota(jnp.int32, _LANES)
    base = wid * _RPW
    rbase = _TC_ROWS + wid * _RSROWS

    pltpu.sync_copy(tgt_hbm.at[pl.ds(base, _RPW)], tgt_v)
    pltpu.sync_copy(tgt_hbm.at[pl.ds(rbase, _RSROWS)], tgt2_v)

    for j in range(_RPW // _LANES):
        t = tgt_v[pl.ds(j * _LANES, _LANES)]
        ivec = base + j * _LANES + lanes
        stripe = ((ivec >> 3) * 250 + (t >> 7)) * 8 + (ivec & 7)
        ridx_v[pl.ds(j * _LANES, _LANES)] = stripe * 8 + ((t >> 4) & 7)
    gather_dma = pltpu.async_copy(x16_hbm.at[ridx_v], rows_v, gsem)

    vbase = rbase * _CHUNKS

    def seg_dma(s, b):
        return pltpu.make_async_copy(
            x16_hbm.at[pl.ds(vbase + s * _SEGV, _SEGV), :], bufs[b], sems[b])

    for b in range(_NBUF):
        seg_dma(b, b).start()

    def seg_body(s, b, carry):
        racc, x0acc = carry
        buf = bufs[b]

        def tile(ti, accs):
            o = ti * 64
            out = []
            for c in range(8):
                a = accs[c]
                for q in range(8):
                    a = a + buf[o + c * 8 + q, :]
                out.append(a)
            return tuple(out)

        z = jnp.zeros((_LANES,), jnp.float32)
        accs = lax.fori_loop(0, _SEGV // 64, tile, (z,) * 8)

        lr0 = (s // _SEG_PER_GROUP) * 8
        tg = tgt2_v[pl.ds((lr0 >> 4) << 4, _LANES)]
        x0f = jnp.where(s % _SEG_PER_GROUP == 0, 1.0, 0.0)
        for c in range(8):
            t_r = _lane_pick(
                tg, jnp.broadcast_to((lr0 + c) & (_LANES - 1), (_LANES,)))
            mf = jnp.minimum(jnp.abs(t_r), 1).astype(jnp.float32)
            racc = racc + mf * accs[c]
            x0acc = x0acc + (mf * x0f) * jnp.where(
                lanes == 0, buf[c * 8, :], 0.0)
        return racc, x0acc

    def group(gi, carry):
        for b in range(_NBUF):
            s = gi * _NBUF + b
            seg_dma(s, b).wait()
            carry = seg_body(s, b, carry)

            @pl.when(s + _NBUF < _NSEG)
            def _():
                seg_dma(s + _NBUF, b).start()
        return carry

    z16 = jnp.zeros((_LANES,), jnp.float32)
    racc, x0acc = lax.fori_loop(0, _NSEG // _NBUF, group, (z16, z16))
    racc_v[...] = _S * x0acc - _S * racc
    pltpu.sync_copy(racc_v, rout_hbm.at[wid])

    gather_dma.wait()
    gacc = jnp.zeros((_LANES,), jnp.float32)
    for c in range(_RPW // _LANES):
        t = tgt_v[pl.ds(c * _LANES, _LANES)]
        lvec = t & (_LANES - 1)
        keep = t != _PAD
        for j in range(_LANES):
            g = _lane_pick(rows_v[c * _LANES + j, :], lvec)
            gacc = gacc + jnp.where((lanes == j) & keep, g, 0.0)
    gacc_v[...] = gacc
    pltpu.sync_copy(gacc_v, gout_hbm.at[wid])


def kernel(x, target, mu, logvar, beta):
    del mu, logvar, beta
    tgt = target.astype(jnp.int32)
    x4 = x.reshape(512, 8, 250, 128).transpose(0, 2, 1, 3).reshape(-1, _LANES)
    g_out, r_out = _sc_kernel(x4, tgt)
    tc_sum, tc_cnt = _tc_call(x, tgt.reshape(_N, 1))
    total = (tc_cnt[0, 0] * _K + tc_sum[0, 0] + jnp.sum(r_out)
             + (_S - _CONF) * jnp.sum(g_out))
    return total / _N


# speedup vs baseline: 6.9201x; 1.0114x over previous
import functools
import math

import jax
import jax.numpy as jnp
from jax import lax
from jax.experimental import pallas as pl
from jax.experimental.pallas import tpu as pltpu
from jax.experimental.pallas import tpu_sc as plsc

_SIZE = 32000
_PAD = 0
_SMOOTHING = 0.1
_CONF = 1.0 - _SMOOTHING
_N = 4096
_S = _SMOOTHING / (_SIZE - 2)
_K = (_SIZE - 2) * _S * math.log(_S) + _CONF * math.log(_CONF)

_LANES = 16
_NC = 2
_NS = 16
_NW = _NC * _NS
_RPW = _N // _NW
_CHUNKS = _SIZE // _LANES

_TC_ROWS = 1536
_RSROWS = (_N - _TC_ROWS) // _NW
_VPW = _RSROWS * _CHUNKS
_SEGV = 1600
_NSEG = _VPW // _SEGV
_SEG_PER_GROUP = 10
_NBUF = 4

_TC_BLOCK_ROWS = 64
_TC_NBUF = 6
_TC_NBLOCKS = _TC_ROWS // _TC_BLOCK_ROWS


def _tc_body(x_hbm, t_ref, sum_ref, cnt_ref, bufs, sems):
    def start(i):
        b = i % _TC_NBUF
        pltpu.make_async_copy(
            x_hbm.at[pl.ds(i * _TC_BLOCK_ROWS, _TC_BLOCK_ROWS), :],
            bufs.at[b], sems.at[b]).start()

    for i in range(_TC_NBUF):
        start(i)

    acc = jnp.float32(0.0)
    for i in range(_TC_NBLOCKS):
        b = i % _TC_NBUF
        pltpu.make_async_copy(
            x_hbm.at[pl.ds(i * _TC_BLOCK_ROWS, _TC_BLOCK_ROWS), :],
            bufs.at[b], sems.at[b]).wait()
        xb = bufs[b]
        t = t_ref[pl.ds(i * _TC_BLOCK_ROWS, _TC_BLOCK_ROWS), :]
        m = t != _PAD
        rs = jnp.sum(xb, axis=1, keepdims=True)
        x0 = xb[:, 0:1]
        acc = acc + jnp.sum(jnp.where(m, _S * x0 - _S * rs, 0.0))
        if i + _TC_NBUF < _TC_NBLOCKS:
            start(i + _TC_NBUF)

    cnt = jnp.sum((t_ref[...] != _PAD).astype(jnp.float32))
    sum_ref[0, 0] = acc
    cnt_ref[0, 0] = cnt


def _tc_call(x, t2d):
    return pl.pallas_call(
        _tc_body,
        in_specs=[
            pl.BlockSpec(memory_space=pl.ANY),
            pl.BlockSpec(memory_space=pltpu.VMEM),
        ],
        out_specs=[
            pl.BlockSpec(memory_space=pltpu.SMEM),
            pl.BlockSpec(memory_space=pltpu.SMEM),
        ],
        out_shape=[
            jax.ShapeDtypeStruct((1, 1), jnp.float32),
            jax.ShapeDtypeStruct((1, 1), jnp.float32),
        ],
        scratch_shapes=[
            pltpu.VMEM((_TC_NBUF, _TC_BLOCK_ROWS, _SIZE), jnp.float32),
            pltpu.SemaphoreType.DMA((_TC_NBUF,)),
        ],
    )(x, t2d)


def _lane_pick(row, lvec):
    return lax.gather(
        row, lvec[:, None],
        lax.GatherDimensionNumbers(
            offset_dims=(), collapsed_slice_dims=(0,),
            start_index_map=(0,)),
        slice_sizes=(1,),
        mode=lax.GatherScatterMode.PROMISE_IN_BOUNDS)


@functools.partial(
    pl.kernel,
    out_type=[
        jax.ShapeDtypeStruct((_NW, _LANES), jnp.float32),
        jax.ShapeDtypeStruct((_NW, _LANES), jnp.float32),
    ],
    mesh=plsc.VectorSubcoreMesh(
        core_axis_name="c", subcore_axis_name="s",
        num_cores=_NC, num_subcores=_NS,
    ),
    compiler_params=pltpu.CompilerParams(use_tc_tiling_on_sc=False),
    scratch_types=[
        pltpu.VMEM((_RPW,), jnp.int32),
        pltpu.VMEM((_RSROWS,), jnp.int32),
        pltpu.VMEM((_RPW,), jnp.int32),
        pltpu.VMEM((_RPW, _LANES), jnp.float32),
        [pltpu.VMEM((_SEGV, _LANES), jnp.float32) for _ in range(_NBUF)],
        pltpu.VMEM((_LANES,), jnp.float32),
        pltpu.VMEM((_LANES,), jnp.float32),
        pltpu.SemaphoreType.DMA,
        [pltpu.SemaphoreType.DMA for _ in range(_NBUF)],
    ],
)
def _sc_kernel(x16_hbm, tgt_hbm, gout_hbm, rout_hbm,
               tgt_v, tgt2_v, ridx_v, rows_v, bufs, gacc_v, racc_v,
               gsem, sems):
    wid = lax.axis_index("s") * _NC + lax.axis_index("c")
    lanes = lax.iota(jnp.int32, _LANES)
    base = wid * _RPW
    rbase = _TC_ROWS + wid * _RSROWS

    pltpu.sync_copy(tgt_hbm.at[pl.ds(base, _RPW)], tgt_v)
    pltpu.sync_copy(tgt_hbm.at[pl.ds(rbase, _RSROWS)], tgt2_v)

    for j in range(_RPW // _LANES):
        t = tgt_v[pl.ds(j * _LANES, _LANES)]
        ivec = base + j * _LANES + lanes
        stripe = ((ivec >> 3) * 250 + (t >> 7)) * 8 + (ivec & 7)
        ridx_v[pl.ds(j * _LANES, _LANES)] = stripe * 8 + ((t >> 4) & 7)
    gather_dma = pltpu.async_copy(x16_hbm.at[ridx_v], rows_v, gsem)

    vbase = rbase * _CHUNKS

    def seg_dma(s, b):
        return pltpu.make_async_copy(
            x16_hbm.at[pl.ds(vbase + s * _SEGV, _SEGV), :], bufs[b], sems[b])

    for b in range(_NBUF):
        seg_dma(b, b).start()

    def seg_body(s, b, carry):
        racc, x0acc = carry
        buf = bufs[b]

        def tile(ti, accs):
            o = ti * 64
            out = []
            for c in range(8):
                a = accs[c]
                for q in range(8):
                    a = a + buf[o + c * 8 + q, :]
                out.append(a)
            return tuple(out)

        z = jnp.zeros((_LANES,), jnp.float32)
        accs = lax.fori_loop(0, _SEGV // 64, tile, (z,) * 8)

        lr0 = (s // _SEG_PER_GROUP) * 8
        tg = tgt2_v[pl.ds((lr0 >> 4) << 4, _LANES)]
        x0f = jnp.where(s % _SEG_PER_GROUP == 0, 1.0, 0.0)
        for c in range(8):
            t_r = _lane_pick(
                tg, jnp.broadcast_to((lr0 + c) & (_LANES - 1), (_LANES,)))
            mf = jnp.minimum(jnp.abs(t_r), 1).astype(jnp.float32)
            racc = racc + mf * accs[c]
            x0acc = x0acc + (mf * x0f) * jnp.where(
                lanes == 0, buf[c * 8, :], 0.0)
        return racc, x0acc

    def group(gi, carry):
        for b in range(_NBUF):
            s = gi * _NBUF + b
            seg_dma(s, b).wait()
            carry = seg_body(s, b, carry)

            @pl.when(s + _NBUF < _NSEG)
            def _():
                seg_dma(s + _NBUF, b).start()
        return carry

    z16 = jnp.zeros((_LANES,), jnp.float32)
    racc, x0acc = lax.fori_loop(0, _NSEG // _NBUF, group, (z16, z16))
    racc_v[...] = _S * x0acc - _S * racc
    pltpu.sync_copy(racc_v, rout_hbm.at[wid])

    gather_dma.wait()
    gacc = jnp.zeros((_LANES,), jnp.float32)
    for c in range(_RPW // _LANES):
        t = tgt_v[pl.ds(c * _LANES, _LANES)]
        lvec = t & (_LANES - 1)
        keep = t != _PAD
        for j in range(_LANES):
            g = _lane_pick(rows_v[c * _LANES + j, :], lvec)
            gacc = gacc + jnp.where((lanes == j) & keep, g, 0.0)
    gacc_v[...] = gacc
    pltpu.sync_copy(gacc_v, gout_hbm.at[wid])


def kernel(x, target, mu, logvar, beta):
    del mu, logvar, beta
    tgt = target.astype(jnp.int32)
    x4 = x.reshape(512, 8, 250, 128).transpose(0, 2, 1, 3).reshape(-1, _LANES)
    g_out, r_out = _sc_kernel(x4, tgt)
    tc_sum, tc_cnt = _tc_call(x, tgt.reshape(_N, 1))
    total = (tc_cnt[0, 0] * _K + tc_sum[0, 0] + jnp.sum(r_out)
             + (_S - _CONF) * jnp.sum(g_out))
    return total / _N


# speedup vs baseline: 7.0048x; 1.0122x over previous
import functools
import math

import jax
import jax.numpy as jnp
from jax import lax
from jax.experimental import pallas as pl
from jax.experimental.pallas import tpu as pltpu
from jax.experimental.pallas import tpu_sc as plsc

_SIZE = 32000
_PAD = 0
_SMOOTHING = 0.1
_CONF = 1.0 - _SMOOTHING
_N = 4096
_S = _SMOOTHING / (_SIZE - 2)
_K = (_SIZE - 2) * _S * math.log(_S) + _CONF * math.log(_CONF)

_LANES = 16
_NC = 2
_NS = 16
_NW = _NC * _NS
_RPW = _N // _NW
_CHUNKS = _SIZE // _LANES

_TC_ROWS = 2048
_RSROWS = (_N - _TC_ROWS) // _NW
_VPW = _RSROWS * _CHUNKS
_SEGV = 1600
_NSEG = _VPW // _SEGV
_SEG_PER_GROUP = 10
_NBUF = 4

_TC_BLOCK_ROWS = 64
_TC_NBUF = 6
_TC_NBLOCKS = _TC_ROWS // _TC_BLOCK_ROWS


def _tc_body(x_hbm, t_ref, sum_ref, cnt_ref, bufs, sems):
    def start(i):
        b = i % _TC_NBUF
        pltpu.make_async_copy(
            x_hbm.at[pl.ds(i * _TC_BLOCK_ROWS, _TC_BLOCK_ROWS), :],
            bufs.at[b], sems.at[b]).start()

    for i in range(_TC_NBUF):
        start(i)

    acc = jnp.float32(0.0)
    for i in range(_TC_NBLOCKS):
        b = i % _TC_NBUF
        pltpu.make_async_copy(
            x_hbm.at[pl.ds(i * _TC_BLOCK_ROWS, _TC_BLOCK_ROWS), :],
            bufs.at[b], sems.at[b]).wait()
        xb = bufs[b]
        t = t_ref[pl.ds(i * _TC_BLOCK_ROWS, _TC_BLOCK_ROWS), :]
        m = t != _PAD
        rs = jnp.sum(xb, axis=1, keepdims=True)
        x0 = xb[:, 0:1]
        acc = acc + jnp.sum(jnp.where(m, _S * x0 - _S * rs, 0.0))
        if i + _TC_NBUF < _TC_NBLOCKS:
            start(i + _TC_NBUF)

    cnt = jnp.sum((t_ref[...] != _PAD).astype(jnp.float32))
    sum_ref[0, 0] = acc
    cnt_ref[0, 0] = cnt


def _tc_call(x, t2d):
    return pl.pallas_call(
        _tc_body,
        in_specs=[
            pl.BlockSpec(memory_space=pl.ANY),
            pl.BlockSpec(memory_space=pltpu.VMEM),
        ],
        out_specs=[
            pl.BlockSpec(memory_space=pltpu.SMEM),
            pl.BlockSpec(memory_space=pltpu.SMEM),
        ],
        out_shape=[
            jax.ShapeDtypeStruct((1, 1), jnp.float32),
            jax.ShapeDtypeStruct((1, 1), jnp.float32),
        ],
        scratch_shapes=[
            pltpu.VMEM((_TC_NBUF, _TC_BLOCK_ROWS, _SIZE), jnp.float32),
            pltpu.SemaphoreType.DMA((_TC_NBUF,)),
        ],
    )(x, t2d)


def _lane_pick(row, lvec):
    return lax.gather(
        row, lvec[:, None],
        lax.GatherDimensionNumbers(
            offset_dims=(), collapsed_slice_dims=(0,),
            start_index_map=(0,)),
        slice_sizes=(1,),
        mode=lax.GatherScatterMode.PROMISE_IN_BOUNDS)


@functools.partial(
    pl.kernel,
    out_type=[
        jax.ShapeDtypeStruct((_NW, _LANES), jnp.float32),
        jax.ShapeDtypeStruct((_NW, _LANES), jnp.float32),
    ],
    mesh=plsc.VectorSubcoreMesh(
        core_axis_name="c", subcore_axis_name="s",
        num_cores=_NC, num_subcores=_NS,
    ),
    compiler_params=pltpu.CompilerParams(use_tc_tiling_on_sc=False),
    scratch_types=[
        pltpu.VMEM((_RPW,), jnp.int32),
        pltpu.VMEM((_RSROWS,), jnp.int32),
        pltpu.VMEM((_RPW,), jnp.int32),
        pltpu.VMEM((_RPW, _LANES), jnp.float32),
        [pltpu.VMEM((_SEGV, _LANES), jnp.float32) for _ in range(_NBUF)],
        pltpu.VMEM((_LANES,), jnp.float32),
        pltpu.VMEM((_LANES,), jnp.float32),
        pltpu.SemaphoreType.DMA,
        [pltpu.SemaphoreType.DMA for _ in range(_NBUF)],
    ],
)
def _sc_kernel(x16_hbm, tgt_hbm, gout_hbm, rout_hbm,
               tgt_v, tgt2_v, ridx_v, rows_v, bufs, gacc_v, racc_v,
               gsem, sems):
    wid = lax.axis_index("s") * _NC + lax.axis_index("c")
    lanes = lax.iota(jnp.int32, _LANES)
    base = wid * _RPW
    rbase = _TC_ROWS + wid * _RSROWS

    pltpu.sync_copy(tgt_hbm.at[pl.ds(base, _RPW)], tgt_v)
    pltpu.sync_copy(tgt_hbm.at[pl.ds(rbase, _RSROWS)], tgt2_v)

    for j in range(_RPW // _LANES):
        t = tgt_v[pl.ds(j * _LANES, _LANES)]
        ivec = base + j * _LANES + lanes
        stripe = ((ivec >> 3) * 250 + (t >> 7)) * 8 + (ivec & 7)
        ridx_v[pl.ds(j * _LANES, _LANES)] = stripe * 8 + ((t >> 4) & 7)
    gather_dma = pltpu.async_copy(x16_hbm.at[ridx_v], rows_v, gsem)

    vbase = rbase * _CHUNKS

    def seg_dma(s, b):
        return pltpu.make_async_copy(
            x16_hbm.at[pl.ds(vbase + s * _SEGV, _SEGV), :], bufs[b], sems[b])

    for b in range(_NBUF):
        seg_dma(b, b).start()

    def seg_body(s, b, carry):
        racc, x0acc = carry
        buf = bufs[b]

        def tile(ti, accs):
            o = ti * 64
            out = []
            for c in range(8):
                a = accs[c]
                for q in range(8):
                    a = a + buf[o + c * 8 + q, :]
                out.append(a)
            return tuple(out)

        z = jnp.zeros((_LANES,), jnp.float32)
        accs = lax.fori_loop(0, _SEGV // 64, tile, (z,) * 8)

        lr0 = (s // _SEG_PER_GROUP) * 8
        tg = tgt2_v[pl.ds((lr0 >> 4) << 4, _LANES)]
        x0f = jnp.where(s % _SEG_PER_GROUP == 0, 1.0, 0.0)
        for c in range(8):
            t_r = _lane_pick(
                tg, jnp.broadcast_to((lr0 + c) & (_LANES - 1), (_LANES,)))
            mf = jnp.minimum(jnp.abs(t_r), 1).astype(jnp.float32)
            racc = racc + mf * accs[c]
            x0acc = x0acc + (mf * x0f) * jnp.where(
                lanes == 0, buf[c * 8, :], 0.0)
        return racc, x0acc

    def group(gi, carry):
        for b in range(_NBUF):
            s = gi * _NBUF + b
            seg_dma(s, b).wait()
            carry = seg_body(s, b, carry)

            @pl.when(s + _NBUF < _NSEG)
            def _():
                seg_dma(s + _NBUF, b).start()
        return carry

    z16 = jnp.zeros((_LANES,), jnp.float32)
    racc, x0acc = lax.fori_loop(0, _NSEG // _NBUF, group, (z16, z16))
    racc_v[...] = _S * x0acc - _S * racc
    pltpu.sync_copy(racc_v, rout_hbm.at[wid])

    gather_dma.wait()
    gacc = jnp.zeros((_LANES,), jnp.float32)
    for c in range(_RPW // _LANES):
        t = tgt_v[pl.ds(c * _LANES, _LANES)]
        lvec = t & (_LANES - 1)
        keep = t != _PAD
        for j in range(_LANES):
            g = _lane_pick(rows_v[c * _LANES + j, :], lvec)
            gacc = gacc + jnp.where((lanes == j) & keep, g, 0.0)
    gacc_v[...] = gacc
    pltpu.sync_copy(gacc_v, gout_hbm.at[wid])


def kernel(x, target, mu, logvar, beta):
    del mu, logvar, beta
    tgt = target.astype(jnp.int32)
    x4 = x.reshape(512, 8, 250, 128).transpose(0, 2, 1, 3).reshape(-1, _LANES)
    g_out, r_out = _sc_kernel(x4, tgt)
    tc_sum, tc_cnt = _tc_call(x, tgt.reshape(_N, 1))
    total = (tc_cnt[0, 0] * _K + tc_sum[0, 0] + jnp.sum(r_out)
             + (_S - _CONF) * jnp.sum(g_out))
    return total / _N


# speedup vs baseline: 7.1688x; 1.0234x over previous
import functools
import math

import jax
import jax.numpy as jnp
from jax import lax
from jax.experimental import pallas as pl
from jax.experimental.pallas import tpu as pltpu
from jax.experimental.pallas import tpu_sc as plsc

_SIZE = 32000
_PAD = 0
_SMOOTHING = 0.1
_CONF = 1.0 - _SMOOTHING
_N = 4096
_S = _SMOOTHING / (_SIZE - 2)
_K = (_SIZE - 2) * _S * math.log(_S) + _CONF * math.log(_CONF)

_LANES = 16
_NC = 2
_NS = 16
_NW = _NC * _NS
_RPW = _N // _NW
_CHUNKS = _SIZE // _LANES

_TC_ROWS = 2560
_RSROWS = (_N - _TC_ROWS) // _NW
_VPW = _RSROWS * _CHUNKS
_SEGV = 1600
_NSEG = _VPW // _SEGV
_SEG_PER_GROUP = 10
_NBUF = 4

_TC_BLOCK_ROWS = 64
_TC_NBUF = 6
_TC_NBLOCKS = _TC_ROWS // _TC_BLOCK_ROWS


def _tc_body(x_hbm, t_ref, sum_ref, cnt_ref, bufs, sems):
    def start(i):
        b = i % _TC_NBUF
        pltpu.make_async_copy(
            x_hbm.at[pl.ds(i * _TC_BLOCK_ROWS, _TC_BLOCK_ROWS), :],
            bufs.at[b], sems.at[b]).start()

    for i in range(_TC_NBUF):
        start(i)

    acc = jnp.float32(0.0)
    for i in range(_TC_NBLOCKS):
        b = i % _TC_NBUF
        pltpu.make_async_copy(
            x_hbm.at[pl.ds(i * _TC_BLOCK_ROWS, _TC_BLOCK_ROWS), :],
            bufs.at[b], sems.at[b]).wait()
        xb = bufs[b]
        t = t_ref[pl.ds(i * _TC_BLOCK_ROWS, _TC_BLOCK_ROWS), :]
        m = t != _PAD
        rs = jnp.sum(xb, axis=1, keepdims=True)
        x0 = xb[:, 0:1]
        acc = acc + jnp.sum(jnp.where(m, _S * x0 - _S * rs, 0.0))
        if i + _TC_NBUF < _TC_NBLOCKS:
            start(i + _TC_NBUF)

    cnt = jnp.sum((t_ref[...] != _PAD).astype(jnp.float32))
    sum_ref[0, 0] = acc
    cnt_ref[0, 0] = cnt


def _tc_call(x, t2d):
    return pl.pallas_call(
        _tc_body,
        in_specs=[
            pl.BlockSpec(memory_space=pl.ANY),
            pl.BlockSpec(memory_space=pltpu.VMEM),
        ],
        out_specs=[
            pl.BlockSpec(memory_space=pltpu.SMEM),
            pl.BlockSpec(memory_space=pltpu.SMEM),
        ],
        out_shape=[
            jax.ShapeDtypeStruct((1, 1), jnp.float32),
            jax.ShapeDtypeStruct((1, 1), jnp.float32),
        ],
        scratch_shapes=[
            pltpu.VMEM((_TC_NBUF, _TC_BLOCK_ROWS, _SIZE), jnp.float32),
            pltpu.SemaphoreType.DMA((_TC_NBUF,)),
        ],
    )(x, t2d)


def _lane_pick(row, lvec):
    return lax.gather(
        row, lvec[:, None],
        lax.GatherDimensionNumbers(
            offset_dims=(), collapsed_slice_dims=(0,),
            start_index_map=(0,)),
        slice_sizes=(1,),
        mode=lax.GatherScatterMode.PROMISE_IN_BOUNDS)


@functools.partial(
    pl.kernel,
    out_type=[
        jax.ShapeDtypeStruct((_NW, _LANES), jnp.float32),
        jax.ShapeDtypeStruct((_NW, _LANES), jnp.float32),
    ],
    mesh=plsc.VectorSubcoreMesh(
        core_axis_name="c", subcore_axis_name="s",
        num_cores=_NC, num_subcores=_NS,
    ),
    compiler_params=pltpu.CompilerParams(use_tc_tiling_on_sc=False),
    scratch_types=[
        pltpu.VMEM((_RPW,), jnp.int32),
        pltpu.VMEM((_RSROWS,), jnp.int32),
        pltpu.VMEM((_RPW,), jnp.int32),
        pltpu.VMEM((_RPW, _LANES), jnp.float32),
        [pltpu.VMEM((_SEGV, _LANES), jnp.float32) for _ in range(_NBUF)],
        pltpu.VMEM((_LANES,), jnp.float32),
        pltpu.VMEM((_LANES,), jnp.float32),
        pltpu.SemaphoreType.DMA,
        [pltpu.SemaphoreType.DMA for _ in range(_NBUF)],
    ],
)
def _sc_kernel(x16_hbm, tgt_hbm, gout_hbm, rout_hbm,
               tgt_v, tgt2_v, ridx_v, rows_v, bufs, gacc_v, racc_v,
               gsem, sems):
    wid = lax.axis_index("s") * _NC + lax.axis_index("c")
    lanes = lax.iota(jnp.int32, _LANES)
    base = wid * _RPW
    rbase = _TC_ROWS + wid * _RSROWS

    pltpu.sync_copy(tgt_hbm.at[pl.ds(base, _RPW)], tgt_v)
    pltpu.sync_copy(tgt_hbm.at[pl.ds(rbase, _RSROWS)], tgt2_v)

    for j in range(_RPW // _LANES):
        t = tgt_v[pl.ds(j * _LANES, _LANES)]
        ivec = base + j * _LANES + lanes
        stripe = ((ivec >> 3) * 250 + (t >> 7)) * 8 + (ivec & 7)
        ridx_v[pl.ds(j * _LANES, _LANES)] = stripe * 8 + ((t >> 4) & 7)
    gather_dma = pltpu.async_copy(x16_hbm.at[ridx_v], rows_v, gsem)

    vbase = rbase * _CHUNKS

    def seg_dma(s, b):
        return pltpu.make_async_copy(
            x16_hbm.at[pl.ds(vbase + s * _SEGV, _SEGV), :], bufs[b], sems[b])

    for b in range(_NBUF):
        seg_dma(b, b).start()

    def seg_body(s, b, carry):
        racc, x0acc = carry
        buf = bufs[b]

        def tile(ti, accs):
            o = ti * 64
            out = []
            for c in range(8):
                a = accs[c]
                for q in range(8):
                    a = a + buf[o + c * 8 + q, :]
                out.append(a)
            return tuple(out)

        z = jnp.zeros((_LANES,), jnp.float32)
        accs = lax.fori_loop(0, _SEGV // 64, tile, (z,) * 8)

        lr0 = (s // _SEG_PER_GROUP) * 8
        tg = tgt2_v[pl.ds((lr0 >> 4) << 4, _LANES)]
        x0f = jnp.where(s % _SEG_PER_GROUP == 0, 1.0, 0.0)
        for c in range(8):
            t_r = _lane_pick(
                tg, jnp.broadcast_to((lr0 + c) & (_LANES - 1), (_LANES,)))
            mf = jnp.minimum(jnp.abs(t_r), 1).astype(jnp.float32)
            racc = racc + mf * accs[c]
            x0acc = x0acc + (mf * x0f) * jnp.where(
                lanes == 0, buf[c * 8, :], 0.0)
        return racc, x0acc

    def group(gi, carry):
        for b in range(_NBUF):
            s = gi * _NBUF + b
            seg_dma(s, b).wait()
            carry = seg_body(s, b, carry)

            @pl.when(s + _NBUF < _NSEG)
            def _():
                seg_dma(s + _NBUF, b).start()
        return carry

    z16 = jnp.zeros((_LANES,), jnp.float32)
    racc, x0acc = lax.fori_loop(0, _NSEG // _NBUF, group, (z16, z16))
    racc_v[...] = _S * x0acc - _S * racc
    pltpu.sync_copy(racc_v, rout_hbm.at[wid])

    gather_dma.wait()
    gacc = jnp.zeros((_LANES,), jnp.float32)
    for c in range(_RPW // _LANES):
        t = tgt_v[pl.ds(c * _LANES, _LANES)]
        lvec = t & (_LANES - 1)
        keep = t != _PAD
        for j in range(_LANES):
            g = _lane_pick(rows_v[c * _LANES + j, :], lvec)
            gacc = gacc + jnp.where((lanes == j) & keep, g, 0.0)
    gacc_v[...] = gacc
    pltpu.sync_copy(gacc_v, gout_hbm.at[wid])


def kernel(x, target, mu, logvar, beta):
    del mu, logvar, beta
    tgt = target.astype(jnp.int32)
    x4 = x.reshape(512, 8, 250, 128).transpose(0, 2, 1, 3).reshape(-1, _LANES)
    g_out, r_out = _sc_kernel(x4, tgt)
    tc_sum, tc_cnt = _tc_call(x, tgt.reshape(_N, 1))
    total = (tc_cnt[0, 0] * _K + tc_sum[0, 0] + jnp.sum(r_out)
             + (_S - _CONF) * jnp.sum(g_out))
    return total / _N


# speedup vs baseline: 7.3252x; 1.0218x over previous
import functools
import math

import jax
import jax.numpy as jnp
from jax import lax
from jax.experimental import pallas as pl
from jax.experimental.pallas import tpu as pltpu
from jax.experimental.pallas import tpu_sc as plsc

_SIZE = 32000
_PAD = 0
_SMOOTHING = 0.1
_CONF = 1.0 - _SMOOTHING
_N = 4096
_S = _SMOOTHING / (_SIZE - 2)
_K = (_SIZE - 2) * _S * math.log(_S) + _CONF * math.log(_CONF)

_LANES = 16
_NC = 2
_NS = 16
_NW = _NC * _NS
_RPW = _N // _NW
_CHUNKS = _SIZE // _LANES

_TC_ROWS = 3072
_RSROWS = (_N - _TC_ROWS) // _NW
_VPW = _RSROWS * _CHUNKS
_SEGV = 1600
_NSEG = _VPW // _SEGV
_SEG_PER_GROUP = 10
_NBUF = 4

_TC_BLOCK_ROWS = 64
_TC_NBUF = 6
_TC_NBLOCKS = _TC_ROWS // _TC_BLOCK_ROWS


def _tc_body(x_hbm, t_ref, sum_ref, cnt_ref, bufs, sems):
    def start(i):
        b = i % _TC_NBUF
        pltpu.make_async_copy(
            x_hbm.at[pl.ds(i * _TC_BLOCK_ROWS, _TC_BLOCK_ROWS), :],
            bufs.at[b], sems.at[b]).start()

    for i in range(_TC_NBUF):
        start(i)

    acc = jnp.float32(0.0)
    for i in range(_TC_NBLOCKS):
        b = i % _TC_NBUF
        pltpu.make_async_copy(
            x_hbm.at[pl.ds(i * _TC_BLOCK_ROWS, _TC_BLOCK_ROWS), :],
            bufs.at[b], sems.at[b]).wait()
        xb = bufs[b]
        t = t_ref[pl.ds(i * _TC_BLOCK_ROWS, _TC_BLOCK_ROWS), :]
        m = t != _PAD
        rs = jnp.sum(xb, axis=1, keepdims=True)
        x0 = xb[:, 0:1]
        acc = acc + jnp.sum(jnp.where(m, _S * x0 - _S * rs, 0.0))
        if i + _TC_NBUF < _TC_NBLOCKS:
            start(i + _TC_NBUF)

    cnt = jnp.sum((t_ref[...] != _PAD).astype(jnp.float32))
    sum_ref[0, 0] = acc
    cnt_ref[0, 0] = cnt


def _tc_call(x, t2d):
    return pl.pallas_call(
        _tc_body,
        in_specs=[
            pl.BlockSpec(memory_space=pl.ANY),
            pl.BlockSpec(memory_space=pltpu.VMEM),
        ],
        out_specs=[
            pl.BlockSpec(memory_space=pltpu.SMEM),
            pl.BlockSpec(memory_space=pltpu.SMEM),
        ],
        out_shape=[
            jax.ShapeDtypeStruct((1, 1), jnp.float32),
            jax.ShapeDtypeStruct((1, 1), jnp.float32),
        ],
        scratch_shapes=[
            pltpu.VMEM((_TC_NBUF, _TC_BLOCK_ROWS, _SIZE), jnp.float32),
            pltpu.SemaphoreType.DMA((_TC_NBUF,)),
        ],
    )(x, t2d)


def _lane_pick(row, lvec):
    return lax.gather(
        row, lvec[:, None],
        lax.GatherDimensionNumbers(
            offset_dims=(), collapsed_slice_dims=(0,),
            start_index_map=(0,)),
        slice_sizes=(1,),
        mode=lax.GatherScatterMode.PROMISE_IN_BOUNDS)


@functools.partial(
    pl.kernel,
    out_type=[
        jax.ShapeDtypeStruct((_NW, _LANES), jnp.float32),
        jax.ShapeDtypeStruct((_NW, _LANES), jnp.float32),
    ],
    mesh=plsc.VectorSubcoreMesh(
        core_axis_name="c", subcore_axis_name="s",
        num_cores=_NC, num_subcores=_NS,
    ),
    compiler_params=pltpu.CompilerParams(use_tc_tiling_on_sc=False),
    scratch_types=[
        pltpu.VMEM((_RPW,), jnp.int32),
        pltpu.VMEM((_RSROWS,), jnp.int32),
        pltpu.VMEM((_RPW,), jnp.int32),
        pltpu.VMEM((_RPW, _LANES), jnp.float32),
        [pltpu.VMEM((_SEGV, _LANES), jnp.float32) for _ in range(_NBUF)],
        pltpu.VMEM((_LANES,), jnp.float32),
        pltpu.VMEM((_LANES,), jnp.float32),
        pltpu.SemaphoreType.DMA,
        [pltpu.SemaphoreType.DMA for _ in range(_NBUF)],
    ],
)
def _sc_kernel(x16_hbm, tgt_hbm, gout_hbm, rout_hbm,
               tgt_v, tgt2_v, ridx_v, rows_v, bufs, gacc_v, racc_v,
               gsem, sems):
    wid = lax.axis_index("s") * _NC + lax.axis_index("c")
    lanes = lax.iota(jnp.int32, _LANES)
    base = wid * _RPW
    rbase = _TC_ROWS + wid * _RSROWS

    pltpu.sync_copy(tgt_hbm.at[pl.ds(base, _RPW)], tgt_v)
    pltpu.sync_copy(tgt_hbm.at[pl.ds(rbase, _RSROWS)], tgt2_v)

    for j in range(_RPW // _LANES):
        t = tgt_v[pl.ds(j * _LANES, _LANES)]
        ivec = base + j * _LANES + lanes
        stripe = ((ivec >> 3) * 250 + (t >> 7)) * 8 + (ivec & 7)
        ridx_v[pl.ds(j * _LANES, _LANES)] = stripe * 8 + ((t >> 4) & 7)
    gather_dma = pltpu.async_copy(x16_hbm.at[ridx_v], rows_v, gsem)

    vbase = rbase * _CHUNKS

    def seg_dma(s, b):
        return pltpu.make_async_copy(
            x16_hbm.at[pl.ds(vbase + s * _SEGV, _SEGV), :], bufs[b], sems[b])

    for b in range(_NBUF):
        seg_dma(b, b).start()

    def seg_body(s, b, carry):
        racc, x0acc = carry
        buf = bufs[b]

        def tile(ti, accs):
            o = ti * 64
            out = []
            for c in range(8):
                a = accs[c]
                for q in range(8):
                    a = a + buf[o + c * 8 + q, :]
                out.append(a)
            return tuple(out)

        z = jnp.zeros((_LANES,), jnp.float32)
        accs = lax.fori_loop(0, _SEGV // 64, tile, (z,) * 8)

        lr0 = (s // _SEG_PER_GROUP) * 8
        tg = tgt2_v[pl.ds((lr0 >> 4) << 4, _LANES)]
        x0f = jnp.where(s % _SEG_PER_GROUP == 0, 1.0, 0.0)
        for c in range(8):
            t_r = _lane_pick(
                tg, jnp.broadcast_to((lr0 + c) & (_LANES - 1), (_LANES,)))
            mf = jnp.minimum(jnp.abs(t_r), 1).astype(jnp.float32)
            racc = racc + mf * accs[c]
            x0acc = x0acc + (mf * x0f) * jnp.where(
                lanes == 0, buf[c * 8, :], 0.0)
        return racc, x0acc

    def group(gi, carry):
        for b in range(_NBUF):
            s = gi * _NBUF + b
            seg_dma(s, b).wait()
            carry = seg_body(s, b, carry)

            @pl.when(s + _NBUF < _NSEG)
            def _():
                seg_dma(s + _NBUF, b).start()
        return carry

    z16 = jnp.zeros((_LANES,), jnp.float32)
    racc, x0acc = lax.fori_loop(0, _NSEG // _NBUF, group, (z16, z16))
    racc_v[...] = _S * x0acc - _S * racc
    pltpu.sync_copy(racc_v, rout_hbm.at[wid])

    gather_dma.wait()
    gacc = jnp.zeros((_LANES,), jnp.float32)
    for c in range(_RPW // _LANES):
        t = tgt_v[pl.ds(c * _LANES, _LANES)]
        lvec = t & (_LANES - 1)
        keep = t != _PAD
        for j in range(_LANES):
            g = _lane_pick(rows_v[c * _LANES + j, :], lvec)
            gacc = gacc + jnp.where((lanes == j) & keep, g, 0.0)
    gacc_v[...] = gacc
    pltpu.sync_copy(gacc_v, gout_hbm.at[wid])


def kernel(x, target, mu, logvar, beta):
    del mu, logvar, beta
    tgt = target.astype(jnp.int32)
    x4 = x.reshape(512, 8, 250, 128).transpose(0, 2, 1, 3).reshape(-1, _LANES)
    g_out, r_out = _sc_kernel(x4, tgt)
    tc_sum, tc_cnt = _tc_call(x, tgt.reshape(_N, 1))
    total = (tc_cnt[0, 0] * _K + tc_sum[0, 0] + jnp.sum(r_out)
             + (_S - _CONF) * jnp.sum(g_out))
    return total / _N
